# baseline (speedup 1.0000x reference)
import functools

import jax
import jax.numpy as jnp
from jax import lax
from jax.experimental import pallas as pl
from jax.experimental.pallas import tpu as pltpu
from jax.experimental.pallas import tpu_sc as plsc

N_W = 5000
N_S = 5000
E = 320000
OUT = 128
FEAT = 16

NP = 5120
NCORES = 2
NSUB = 16
NWORK = NCORES * NSUB
EP = 327680
EW = EP // NWORK
CB = 64
NB = EW // CB
DEN_ROWS = 40

NEG_BIG = -1e30


def _dense_body(h_ref, wfcT_ref, wattn_ref, z_ref, s_ref):
    a1 = wattn_ref[0, :OUT]
    z = jnp.dot(h_ref[...], wfcT_ref[...], preferred_element_type=jnp.float32)
    z_ref[...] = z
    s_ref[...] = jnp.sum(z * a1[None, :], axis=1)


def _dense_call(h_p, wfcT, wattn):
    grid = 20
    zb = NP // grid
    return pl.pallas_call(
        _dense_body,
        grid=(grid,),
        in_specs=[
            pl.BlockSpec((zb, OUT), lambda i: (i, 0)),
            pl.BlockSpec((OUT, OUT), lambda i: (0, 0)),
            pl.BlockSpec((1, 3 * OUT), lambda i: (0, 0)),
        ],
        out_specs=[
            pl.BlockSpec((zb, OUT), lambda i: (i, 0)),
            pl.BlockSpec((zb,), lambda i: (i,)),
        ],
        out_shape=[
            jax.ShapeDtypeStruct((NP, OUT), jnp.float32),
            jax.ShapeDtypeStruct((NP,), jnp.float32),
        ],
    )(h_p, wfcT, wattn)


def _edge_body(sd_hbm, tf_hbm, wfeat_hbm, wattn_hbm, ssrc_hbm,
               z_hbm,
               out_hbm, den_hbm,
               s_tab, den_tab, sd_all, srci, dsti, wf_v, a3_v,
               g0, g1, sb0, sb1, tf0, tf1,
               sh_out, sh_den, iota_v,
               gsem0, gsem1, ssem0, ssem1, tsem0, tsem1):
    cid = lax.axis_index("c")
    sid = lax.axis_index("s")
    wid = sid * NCORES + cid
    ebase = wid * EW

    pltpu.sync_copy(ssrc_hbm, s_tab)
    pltpu.sync_copy(sd_hbm.at[pl.ds(ebase, EW)], sd_all)

    pltpu.sync_copy(wattn_hbm.at[0, pl.ds(2 * OUT, OUT)], a3_v)
    w3 = jnp.zeros((16,), jnp.float32)
    for h in range(2):
        pltpu.sync_copy(wfeat_hbm.at[pl.ds(h * 64, 64)], wf_v)

        def _wloop(l8, acc, h=h):
            a16 = a3_v[pl.ds(h * 64 + l8 * 16, 16)]
            for t in range(16):
                acc = acc + jnp.full((16,), a16[t], jnp.float32) \
                    * wf_v[l8 * 16 + t, :]
            return acc
        w3 = lax.fori_loop(0, 4, _wloop, w3)

    zero16 = jnp.zeros((16,), jnp.float32)

    def _zero_den(r, _):
        for j in range(8):
            den_tab[r, pl.ds(j * 16, 16)] = zero16
        return 0
    lax.fori_loop(0, DEN_ROWS, _zero_den, 0)

    def _zero_rows(i, _):
        for j in range(8):
            g0[i, pl.ds(j * 16, 16)] = zero16
        return 0
    lax.fori_loop(0, CB, _zero_rows, 0)

    r0 = sid * (NP // NSUB)
    for c in range((NP // NSUB) // CB):
        pltpu.sync_copy(g0, sh_out.at[pl.ds(r0 + c * CB, CB)])

    @pl.when(sid == 0)
    def _():
        pltpu.sync_copy(den_tab, sh_den)

    ii = lax.iota(jnp.int32, 16)
    iota_v[0, pl.ds(0, 16)] = ii
    iota_v[0, pl.ds(16, 16)] = ii + 16
    iota_v[0, pl.ds(24, 16)] = ii + 24

    gbufs = (g0, g1)
    sbufs = (sb0, sb1)
    tbufs = (tf0, tf1)
    gsems = (gsem0, gsem1)
    ssems = (ssem0, ssem1)
    tsems = (tsem0, tsem1)

    def _tf_base(b):
        return jnp.minimum(ebase + b * CB, E - CB)

    def _start_gather(j, b):
        for g in range(CB // 16):
            v16 = sd_all[pl.ds(b * CB + g * 16, 16)]
            srci[j, pl.ds(g * 16, 16)] = lax.bitwise_and(v16, 0xFFFF)
        pltpu.async_copy(
            z_hbm.at[srci.at[j]], gbufs[j], gsems[j])
        pltpu.async_copy(
            tf_hbm.at[pl.ds(_tf_base(b), CB)], tbufs[j], tsems[j])

    mask0 = lax.iota(jnp.int32, 16) == 0

    def _scale(j, b):
        gb, sb, tb = gbufs[j], sbufs[j], tbufs[j]

        def body(g, _):
            base16 = b * CB + g * 16
            v16 = sd_all[pl.ds(base16, 16)]
            idx16 = lax.bitwise_and(v16, 0xFFFF)
            s16 = plsc.load_gather(s_tab, [idx16])
            d16 = lax.shift_right_logical(v16, 16)
            dsti[j, pl.ds(g * 16, 16)] = d16
            r16 = lax.shift_right_logical(d16, 7)
            c16 = lax.bitwise_and(d16, 127)
            for l in range(16):
                i = g * 16 + l
                fl = jnp.sum(tb[i, :] * w3)
                x = s16[l] + fl
                x = jnp.where(ebase + base16 + l < E, x, NEG_BIG)
                xv = jnp.full((16,), x, jnp.float32)
                exv = jnp.exp(jnp.maximum(xv, xv * 0.01))
                plsc.addupdate_scatter(
                    den_tab,
                    [jnp.full((16,), r16[l], jnp.int32),
                     jnp.full((16,), c16[l], jnp.int32)],
                    exv, mask=mask0)
                for jj in range(8):
                    sl2 = pl.ds(jj * 16, 16)
                    sb[i, sl2] = gb[i, sl2] * exv
            return 0
        lax.fori_loop(0, CB // 16, body, 0)

    _start_gather(0, 0)
    _start_gather(1, 1)

    def _pair(k, _):
        for j in range(2):
            b = 2 * k + j
            pltpu.make_async_copy(
                z_hbm.at[srci.at[j]], gbufs[j], gsems[j]).wait()
            pltpu.make_async_copy(
                tf_hbm.at[pl.ds(_tf_base(b), CB)], tbufs[j], tsems[j]).wait()
            @pl.when(k > 0)
            def _():
                pltpu.make_async_copy(
                    sbufs[j], sh_out.at[dsti.at[j]], ssems[j]).wait()
            _scale(j, b)
            pltpu.async_copy(
                sbufs[j], sh_out.at[dsti.at[j]], ssems[j], add=True)
            @pl.when(b + 2 < NB)
            def _():
                _start_gather(j, b + 2)
        return 0
    with jax.named_scope("sc_rowpipe"):
        lax.fori_loop(0, NB // 2, _pair, 0)

        for j in range(2):
            pltpu.make_async_copy(
                sbufs[j], sh_out.at[dsti.at[j]], ssems[j]).wait()

    plsc.subcore_barrier()

    pltpu.sync_copy(den_tab, sh_den.at[iota_v.at[0]], add=True)

    plsc.subcore_barrier()

    pltpu.sync_copy(sh_out.at[pl.ds(r0, NP // NSUB)],
                    out_hbm.at[cid, pl.ds(r0, NP // NSUB)])

    @pl.when(sid == 0)
    def _():
        pltpu.sync_copy(sh_den, den_hbm.at[cid])


def _edge_call(sd_p, tfidf, wfeat, wattn, ssrc, z):
    mesh = plsc.VectorSubcoreMesh(core_axis_name="c", subcore_axis_name="s")
    fn = pl.kernel(
        _edge_body,
        out_type=[
            jax.ShapeDtypeStruct((NCORES, NP, OUT), jnp.float32),
            jax.ShapeDtypeStruct((NCORES, DEN_ROWS, 128), jnp.float32),
        ],
        mesh=mesh,
        scratch_types=[
            pltpu.VMEM((NP,), jnp.float32),
            pltpu.VMEM((DEN_ROWS, 128), jnp.float32),
            pltpu.VMEM((EW,), jnp.int32),
            pltpu.VMEM((2, CB), jnp.int32),
            pltpu.VMEM((2, CB), jnp.int32),
            pltpu.VMEM((OUT // 2, FEAT), jnp.float32),
            pltpu.VMEM((OUT,), jnp.float32),
            pltpu.VMEM((CB, OUT), jnp.float32),
            pltpu.VMEM((CB, OUT), jnp.float32),
            pltpu.VMEM((CB, OUT), jnp.float32),
            pltpu.VMEM((CB, OUT), jnp.float32),
            pltpu.VMEM((CB, FEAT), jnp.float32),
            pltpu.VMEM((CB, FEAT), jnp.float32),
            pltpu.VMEM_SHARED((NP, OUT), jnp.float32),
            pltpu.VMEM_SHARED((DEN_ROWS, 128), jnp.float32),
            pltpu.VMEM((1, DEN_ROWS), jnp.int32),
            pltpu.SemaphoreType.DMA,
            pltpu.SemaphoreType.DMA,
            pltpu.SemaphoreType.DMA,
            pltpu.SemaphoreType.DMA,
            pltpu.SemaphoreType.DMA,
            pltpu.SemaphoreType.DMA,
        ],
        compiler_params=pltpu.CompilerParams(
            needs_layout_passes=False, use_tc_tiling_on_sc=True),
    )
    return fn(sd_p, tfidf, wfeat, wattn, ssrc, z)


def _norm_body(p_ref, d_ref, o_ref):
    p = p_ref[0] + p_ref[1]
    d = d_ref[0] + d_ref[1]
    d = jnp.where(d > 0.0, d, 1.0)
    o_ref[...] = p * (1.0 / d)[:, None]


def _norm_call(outp, den):
    grid = 20
    rb = NP // grid
    return pl.pallas_call(
        _norm_body,
        grid=(grid,),
        in_specs=[
            pl.BlockSpec((NCORES, rb, OUT), lambda i: (0, i, 0)),
            pl.BlockSpec((NCORES, rb), lambda i: (0, i)),
        ],
        out_specs=pl.BlockSpec((rb, OUT), lambda i: (i, 0)),
        out_shape=jax.ShapeDtypeStruct((NP, OUT), jnp.float32),
    )(outp, den)


def kernel(h, edge_index, tfidfembed, W_fc, W_feat, W_attn):
    src = edge_index[0]
    dst = edge_index[1]
    h_p = jnp.pad(h[:N_W], ((0, NP - N_W), (0, 0)))
    z, ssrc = _dense_call(h_p, W_fc.T, W_attn)
    pad_ids = jnp.arange(EP - E, dtype=jnp.int32)
    src_p = jnp.concatenate([src, pad_ids % N_W])
    dst_p = jnp.concatenate([dst, N_S + pad_ids % (NP - N_S)])
    sd_p = jnp.bitwise_or(jnp.left_shift(dst_p, 16), src_p)
    outp, denp = _edge_call(sd_p, tfidfembed, W_feat, W_attn, ssrc, z)
    den = denp.reshape(NCORES, DEN_ROWS * 128)[:, :NP]
    res = _norm_call(outp, den)
    return res[:N_S]


# speedup vs baseline: 1.4291x; 1.4291x over previous
import functools

import jax
import jax.numpy as jnp
from jax import lax
from jax.experimental import pallas as pl
from jax.experimental.pallas import tpu as pltpu
from jax.experimental.pallas import tpu_sc as plsc

N_W = 5000
N_S = 5000
E = 320000
OUT = 128
FEAT = 16

NP = 5120
NCORES = 2
NSUB = 16
NWORK = NCORES * NSUB
EP = 327680
EW = EP // NWORK
CB = 64
NB = EW // CB
DEN_ROWS = 40

NEG_BIG = -1e30


def _dense_body(h_ref, wfcT_ref, wattn_ref, z_ref, s_ref):
    a1 = wattn_ref[0, :OUT]
    z = jnp.dot(h_ref[...], wfcT_ref[...], preferred_element_type=jnp.float32)
    z_ref[...] = z
    s_ref[...] = jnp.sum(z * a1[None, :], axis=1)


def _dense_call(h_p, wfcT, wattn):
    grid = 20
    zb = NP // grid
    return pl.pallas_call(
        _dense_body,
        grid=(grid,),
        in_specs=[
            pl.BlockSpec((zb, OUT), lambda i: (i, 0)),
            pl.BlockSpec((OUT, OUT), lambda i: (0, 0)),
            pl.BlockSpec((1, 3 * OUT), lambda i: (0, 0)),
        ],
        out_specs=[
            pl.BlockSpec((zb, OUT), lambda i: (i, 0)),
            pl.BlockSpec((zb,), lambda i: (i,)),
        ],
        out_shape=[
            jax.ShapeDtypeStruct((NP, OUT), jnp.float32),
            jax.ShapeDtypeStruct((NP,), jnp.float32),
        ],
    )(h_p, wfcT, wattn)


def _edge_body(sd_hbm, tf_hbm, wfeat_hbm, wattn_hbm, ssrc_hbm,
               z_hbm,
               out_hbm, den_hbm,
               s_tab, den_tab, sd_all, srci, dsti, wf_v, a3_v,
               g0, g1, sb0, sb1, tf0, tf1,
               sh_out, sh_den, iota_v,
               gsem0, gsem1, ssem0, ssem1, tsem0, tsem1):
    cid = lax.axis_index("c")
    sid = lax.axis_index("s")
    wid = sid * NCORES + cid
    ebase = wid * EW

    pltpu.sync_copy(ssrc_hbm, s_tab)
    pltpu.sync_copy(sd_hbm.at[pl.ds(ebase, EW)], sd_all)

    pltpu.sync_copy(wattn_hbm.at[0, pl.ds(2 * OUT, OUT)], a3_v)
    w3 = jnp.zeros((16,), jnp.float32)
    for h in range(2):
        pltpu.sync_copy(wfeat_hbm.at[pl.ds(h * 64, 64)], wf_v)

        def _wloop(l8, acc, h=h):
            a16 = a3_v[pl.ds(h * 64 + l8 * 16, 16)]
            for t in range(16):
                acc = acc + jnp.full((16,), a16[t], jnp.float32) \
                    * wf_v[l8 * 16 + t, :]
            return acc
        w3 = lax.fori_loop(0, 4, _wloop, w3)

    zero16 = jnp.zeros((16,), jnp.float32)

    def _zero_den(r, _):
        for j in range(8):
            den_tab[r, pl.ds(j * 16, 16)] = zero16
        return 0
    lax.fori_loop(0, DEN_ROWS, _zero_den, 0)

    def _zero_rows(i, _):
        for j in range(8):
            g0[i, pl.ds(j * 16, 16)] = zero16
        return 0
    lax.fori_loop(0, CB, _zero_rows, 0)

    r0 = sid * (NP // NSUB)
    for c in range((NP // NSUB) // CB):
        pltpu.sync_copy(g0, sh_out.at[pl.ds(r0 + c * CB, CB)])

    @pl.when(sid == 0)
    def _():
        pltpu.sync_copy(den_tab, sh_den)

    ii = lax.iota(jnp.int32, 16)
    iota_v[0, pl.ds(0, 16)] = ii
    iota_v[0, pl.ds(16, 16)] = ii + 16
    iota_v[0, pl.ds(24, 16)] = ii + 24

    gbufs = (g0, g1)
    sbufs = (sb0, sb1)
    tbufs = (tf0, tf1)
    gsems = (gsem0, gsem1)
    ssems = (ssem0, ssem1)
    tsems = (tsem0, tsem1)

    def _tf_base(b):
        return jnp.minimum(ebase + b * CB, E - CB)

    def _start_gather(j, b):
        for g in range(CB // 16):
            v16 = sd_all[pl.ds(b * CB + g * 16, 16)]
            srci[j, pl.ds(g * 16, 16)] = lax.bitwise_and(v16, 0xFFFF)
        pltpu.async_copy(
            z_hbm.at[srci.at[j]], gbufs[j], gsems[j])
        pltpu.async_copy(
            tf_hbm.at[pl.ds(_tf_base(b), CB)], tbufs[j], tsems[j])

    ii = lax.iota(jnp.int32, 16)
    cols = [jnp.full((16,), c, jnp.int32) for c in range(FEAT)]
    w3s = [jnp.full((16,), w3[c], jnp.float32) for c in range(FEAT)]

    def _scale(j, b):
        gb, sb, tb = gbufs[j], sbufs[j], tbufs[j]

        def body(g, _):
            base16 = b * CB + g * 16
            v16 = sd_all[pl.ds(base16, 16)]
            idx16 = lax.bitwise_and(v16, 0xFFFF)
            s16 = plsc.load_gather(s_tab, [idx16])
            d16 = lax.shift_right_logical(v16, 16)
            dsti[j, pl.ds(g * 16, 16)] = d16
            r16 = lax.shift_right_logical(d16, 7)
            c16 = lax.bitwise_and(d16, 127)
            ir = ii + g * 16
            f16 = jnp.zeros((16,), jnp.float32)
            for c in range(FEAT):
                f16 = f16 + plsc.load_gather(tb, [ir, cols[c]]) * w3s[c]
            x16 = s16 + f16
            x16 = jnp.where(ebase + base16 + ii < E, x16, NEG_BIG)
            ex16 = jnp.exp(jnp.maximum(x16, x16 * 0.01))
            plsc.addupdate_scatter(den_tab, [r16, c16], ex16)
            for l in range(16):
                i = g * 16 + l
                vx = jnp.full((16,), ex16[l], jnp.float32)
                for jj in range(8):
                    sl2 = pl.ds(jj * 16, 16)
                    sb[i, sl2] = gb[i, sl2] * vx
            return 0
        lax.fori_loop(0, CB // 16, body, 0)

    _start_gather(0, 0)
    _start_gather(1, 1)

    def _pair(k, _):
        for j in range(2):
            b = 2 * k + j
            pltpu.make_async_copy(
                z_hbm.at[srci.at[j]], gbufs[j], gsems[j]).wait()
            pltpu.make_async_copy(
                tf_hbm.at[pl.ds(_tf_base(b), CB)], tbufs[j], tsems[j]).wait()
            @pl.when(k > 0)
            def _():
                pltpu.make_async_copy(
                    sbufs[j], sh_out.at[dsti.at[j]], ssems[j]).wait()
            _scale(j, b)
            pltpu.async_copy(
                sbufs[j], sh_out.at[dsti.at[j]], ssems[j], add=True)
            @pl.when(b + 2 < NB)
            def _():
                _start_gather(j, b + 2)
        return 0
    with jax.named_scope("sc_rowpipe"):
        lax.fori_loop(0, NB // 2, _pair, 0)

        for j in range(2):
            pltpu.make_async_copy(
                sbufs[j], sh_out.at[dsti.at[j]], ssems[j]).wait()

    plsc.subcore_barrier()

    pltpu.sync_copy(den_tab, sh_den.at[iota_v.at[0]], add=True)

    plsc.subcore_barrier()

    pltpu.sync_copy(sh_out.at[pl.ds(r0, NP // NSUB)],
                    out_hbm.at[cid, pl.ds(r0, NP // NSUB)])

    @pl.when(sid == 0)
    def _():
        pltpu.sync_copy(sh_den, den_hbm.at[cid])


def _edge_call(sd_p, tfidf, wfeat, wattn, ssrc, z):
    mesh = plsc.VectorSubcoreMesh(core_axis_name="c", subcore_axis_name="s")
    fn = pl.kernel(
        _edge_body,
        out_type=[
            jax.ShapeDtypeStruct((NCORES, NP, OUT), jnp.float32),
            jax.ShapeDtypeStruct((NCORES, DEN_ROWS, 128), jnp.float32),
        ],
        mesh=mesh,
        scratch_types=[
            pltpu.VMEM((NP,), jnp.float32),
            pltpu.VMEM((DEN_ROWS, 128), jnp.float32),
            pltpu.VMEM((EW,), jnp.int32),
            pltpu.VMEM((2, CB), jnp.int32),
            pltpu.VMEM((2, CB), jnp.int32),
            pltpu.VMEM((OUT // 2, FEAT), jnp.float32),
            pltpu.VMEM((OUT,), jnp.float32),
            pltpu.VMEM((CB, OUT), jnp.float32),
            pltpu.VMEM((CB, OUT), jnp.float32),
            pltpu.VMEM((CB, OUT), jnp.float32),
            pltpu.VMEM((CB, OUT), jnp.float32),
            pltpu.VMEM((CB, FEAT), jnp.float32),
            pltpu.VMEM((CB, FEAT), jnp.float32),
            pltpu.VMEM_SHARED((NP, OUT), jnp.float32),
            pltpu.VMEM_SHARED((DEN_ROWS, 128), jnp.float32),
            pltpu.VMEM((1, DEN_ROWS), jnp.int32),
            pltpu.SemaphoreType.DMA,
            pltpu.SemaphoreType.DMA,
            pltpu.SemaphoreType.DMA,
            pltpu.SemaphoreType.DMA,
            pltpu.SemaphoreType.DMA,
            pltpu.SemaphoreType.DMA,
        ],
        compiler_params=pltpu.CompilerParams(
            needs_layout_passes=False, use_tc_tiling_on_sc=True),
    )
    return fn(sd_p, tfidf, wfeat, wattn, ssrc, z)


def _norm_body(p_ref, d_ref, o_ref):
    p = p_ref[0] + p_ref[1]
    d = d_ref[0] + d_ref[1]
    d = jnp.where(d > 0.0, d, 1.0)
    o_ref[...] = p * (1.0 / d)[:, None]


def _norm_call(outp, den):
    grid = 20
    rb = NP // grid
    return pl.pallas_call(
        _norm_body,
        grid=(grid,),
        in_specs=[
            pl.BlockSpec((NCORES, rb, OUT), lambda i: (0, i, 0)),
            pl.BlockSpec((NCORES, rb), lambda i: (0, i)),
        ],
        out_specs=pl.BlockSpec((rb, OUT), lambda i: (i, 0)),
        out_shape=jax.ShapeDtypeStruct((NP, OUT), jnp.float32),
    )(outp, den)


def kernel(h, edge_index, tfidfembed, W_fc, W_feat, W_attn):
    src = edge_index[0]
    dst = edge_index[1]
    h_p = jnp.pad(h[:N_W], ((0, NP - N_W), (0, 0)))
    z, ssrc = _dense_call(h_p, W_fc.T, W_attn)
    pad_ids = jnp.arange(EP - E, dtype=jnp.int32)
    src_p = jnp.concatenate([src, pad_ids % N_W])
    dst_p = jnp.concatenate([dst, N_S + pad_ids % (NP - N_S)])
    sd_p = jnp.bitwise_or(jnp.left_shift(dst_p, 16), src_p)
    outp, denp = _edge_call(sd_p, tfidfembed, W_feat, W_attn, ssrc, z)
    den = denp.reshape(NCORES, DEN_ROWS * 128)[:, :NP]
    res = _norm_call(outp, den)
    return res[:N_S]
